# baseline (speedup 1.0000x reference)
import functools

import jax
import jax.numpy as jnp
from jax import lax
from jax.experimental import pallas as pl
from jax.experimental.pallas import tpu as pltpu
from jax.experimental.pallas import tpu_sc as plsc

B = 16384
D = 64
NC = 2
NS = 16
NW = NC * NS
BPW = B // NW
CHUNK = 256
NCHUNK = BPW // CHUNK

_mesh = plsc.VectorSubcoreMesh(core_axis_name="c", subcore_axis_name="s")


@functools.partial(
    pl.kernel,
    mesh=_mesh,
    out_type=jax.ShapeDtypeStruct((B, 4 * D), jnp.float32),
    compiler_params=pltpu.CompilerParams(use_tc_tiling_on_sc=False),
    scratch_types=[
        pltpu.VMEM((BPW,), jnp.int32),
        pltpu.VMEM((BPW,), jnp.int32),
        pltpu.VMEM((CHUNK, D), jnp.float32),
        pltpu.VMEM((CHUNK, D), jnp.float32),
        pltpu.VMEM((CHUNK, D), jnp.float32),
        pltpu.VMEM((CHUNK, D), jnp.float32),
        pltpu.SemaphoreType.DMA,
        pltpu.SemaphoreType.DMA,
        pltpu.SemaphoreType.DMA,
        pltpu.SemaphoreType.DMA,
    ],
)
def _gas_concat(adj4_hbm, adj5_hbm, cv0_hbm, cv1_hbm, cv2_hbm, cv3_hbm,
                out_hbm, idx5_v, idx4_v, ri_v, ru_v, c0_v, c3_v,
                sem_ri, sem_ru, sem_c0, sem_c3):
    wid = lax.axis_index("s") * NC + lax.axis_index("c")
    base = wid * BPW

    pltpu.sync_copy(adj5_hbm.at[pl.ds(base, BPW)], idx5_v)
    pltpu.sync_copy(adj4_hbm.at[pl.ds(base, BPW)], idx4_v)

    for c in range(NCHUNK):
        rows = pl.ds(base + c * CHUNK, CHUNK)
        idx_sl = pl.ds(c * CHUNK, CHUNK)
        ri_cp = pltpu.async_copy(cv2_hbm.at[idx5_v.at[idx_sl]], ri_v, sem_ri)
        ru_cp = pltpu.async_copy(cv1_hbm.at[idx4_v.at[idx_sl]], ru_v, sem_ru)
        c0_cp = pltpu.async_copy(cv0_hbm.at[rows], c0_v, sem_c0)
        c3_cp = pltpu.async_copy(cv3_hbm.at[rows], c3_v, sem_c3)
        ri_cp.wait()
        pltpu.sync_copy(ri_v, out_hbm.at[rows, pl.ds(0, D)])
        c0_cp.wait()
        pltpu.sync_copy(c0_v, out_hbm.at[rows, pl.ds(D, D)])
        ru_cp.wait()
        pltpu.sync_copy(ru_v, out_hbm.at[rows, pl.ds(2 * D, D)])
        c3_cp.wait()
        pltpu.sync_copy(c3_v, out_hbm.at[rows, pl.ds(3 * D, D)])


def kernel(adj0, adj1, adj2, adj3, adj4, adj5, cv0, cv1, cv2, cv3):
    return _gas_concat(adj4, adj5, cv0, cv1, cv2, cv3)
